# baseline (speedup 1.0000x reference)
import jax
import jax.numpy as jnp
from jax import lax
from jax.experimental import pallas as pl
from jax.experimental.pallas import tpu as pltpu
from jax.experimental.pallas import tpu_sc as plsc

NUM_ROWS = 1000000
DIM = 32
SEQ = 16384
SLEN = 50
NC, NS = 2, 16
NW = NC * NS
SENT_PER_W = SEQ // NW
TBLK = 4
NBUF = 4
UNITS = TBLK * SLEN
NROUNDS = UNITS // NBUF


def _embed_body(idx_hbm, table_hbm, out_hbm, idx_v, idxt_v, rows_v, tbuf_v,
                gsem, ssem):
    wid = lax.axis_index("s") * NC + lax.axis_index("c")
    s0 = wid * SENT_PER_W

    pltpu.sync_copy(idx_hbm.at[pl.ds(s0, SENT_PER_W)], idx_v)

    lane = lax.iota(jnp.int32, 16)

    @pl.loop(0, SLEN)
    def _build(j):
        col = jnp.full((16,), 0, jnp.int32) + j
        for tt in range(TBLK):
            for m in range(8):
                rows = lane + (128 * tt + 16 * m)
                vals = plsc.load_gather(idx_v, [rows, col])
                idxt_v[tt, j, pl.ds(16 * m, 16)] = vals

    def gather_start(tt, j, b):
        pltpu.async_copy(table_hbm.at[idxt_v.at[tt, j]], rows_v.at[b],
                         gsem.at[b])

    def gather_wait(b):
        pltpu.make_async_copy(table_hbm.at[idxt_v.at[0, 0]], rows_v.at[b],
                              gsem.at[b]).wait()

    def store_start(tt, j, b):
        for a in range(4):
            pltpu.async_copy(tbuf_v.at[b, pl.ds(8 * a, 8)],
                             out_hbm.at[j, a, TBLK * wid + tt], ssem.at[b])

    def store_wait(b):
        for a in range(4):
            pltpu.make_async_copy(tbuf_v.at[b, pl.ds(8 * a, 8)],
                                  out_hbm.at[0, a, 0], ssem.at[b]).wait()

    def unit(u):
        tt = u // SLEN
        j = u - tt * SLEN
        return tt, j

    for b in range(NBUF):
        tt, j = unit(b)
        gather_start(tt, j, b)

    @pl.loop(0, NROUNDS)
    def _round(r):
        for b in range(NBUF):
            u = r * NBUF + b
            tt, j = unit(u)
            gather_wait(b)

            @pl.when(u >= NBUF)
            def _():
                store_wait(b)

            @pl.loop(0, DIM, unroll=8)
            def _tr(k):
                kc = (k + lane) & (DIM - 1)
                for m in range(8):
                    rows = lane + 16 * m
                    vals = plsc.load_gather(rows_v.at[b], [rows, kc])
                    plsc.store_scatter(tbuf_v.at[b], [kc, rows], vals)

            store_start(tt, j, b)
            nxt = u + NBUF

            @pl.when(nxt < UNITS)
            def _():
                tt2 = nxt // SLEN
                j2 = nxt - tt2 * SLEN
                gather_start(tt2, j2, b)

    for b in range(NBUF):
        store_wait(b)


def _wt_body(x_ref, o_ref):
    x = x_ref[...]
    for j in range(4):
        lanes = jnp.broadcast_to(
            jnp.arange(j, 128, 4, dtype=jnp.int32)[None, :], (32, 32)
        )
        sel = jnp.take_along_axis(x, lanes, axis=1)
        o_ref[:, 32 * j:32 * (j + 1)] = sel.T


def _to_row_major(weight):
    wt = weight.T
    w128 = pl.pallas_call(
        _wt_body,
        grid=(7813,),
        in_specs=[pl.BlockSpec((32, 128), lambda g: (0, g))],
        out_specs=pl.BlockSpec((32, 128), lambda g: (g, 0)),
        out_shape=jax.ShapeDtypeStruct((250000, 128), jnp.float32),
    )(wt)
    return w128.reshape(NUM_ROWS, DIM)


@jax.jit
def _embed(idx, weight):
    mesh = plsc.VectorSubcoreMesh(core_axis_name="c", subcore_axis_name="s")
    run = pl.kernel(
        _embed_body,
        out_type=jax.ShapeDtypeStruct((SLEN, 4, SEQ // 128, 8, 128),
                                      jnp.float32),
        mesh=mesh,
        compiler_params=pltpu.CompilerParams(
            use_tc_tiling_on_sc=False, needs_layout_passes=False
        ),
        scratch_types=[
            pltpu.VMEM((SENT_PER_W, SLEN), jnp.int32),
            pltpu.VMEM((TBLK, SLEN, 128), jnp.int32),
            pltpu.VMEM((NBUF, 128, DIM), jnp.float32),
            pltpu.VMEM((NBUF, DIM, 128), jnp.float32),
            pltpu.SemaphoreType.DMA((NBUF,)),
            pltpu.SemaphoreType.DMA((NBUF,)),
        ],
    )
    ot = run(idx, _to_row_major(weight))
    return ot.transpose(2, 4, 0, 1, 3).reshape(SEQ, SLEN, DIM)


def kernel(input, weight):
    return _embed(input.astype(jnp.int32), weight)


# speedup vs baseline: 4.3219x; 4.3219x over previous
import jax
import jax.numpy as jnp
from jax import lax
from jax.experimental import pallas as pl
from jax.experimental.pallas import tpu as pltpu
from jax.experimental.pallas import tpu_sc as plsc

NUM_ROWS = 1000000
DIM = 32
SEQ = 16384
SLEN = 50
NC, NS = 2, 16
NW = NC * NS
SENT_PER_W = SEQ // NW
TBLK = 4
NBUF = 4
UNITS = TBLK * SLEN
NROUNDS = UNITS // NBUF


def _embed_body(idx_hbm, table_hbm, out_hbm, idx_v, idxt_v, rows_v, tbuf_v,
                gsem, ssem):
    wid = lax.axis_index("s") * NC + lax.axis_index("c")
    s0 = wid * SENT_PER_W

    pltpu.sync_copy(idx_hbm.at[pl.ds(s0, SENT_PER_W)], idx_v)

    lane = lax.iota(jnp.int32, 16)

    @pl.loop(0, SLEN)
    def _build(j):
        col = jnp.full((16,), 0, jnp.int32) + j
        for tt in range(TBLK):
            for m in range(8):
                rows = lane + (128 * tt + 16 * m)
                vals = plsc.load_gather(idx_v, [rows, col])
                idxt_v[tt, j, pl.ds(16 * m, 16)] = vals

    def gather_start(tt, j, b):
        pltpu.async_copy(table_hbm.at[idxt_v.at[tt, j]], rows_v.at[b],
                         gsem.at[b])

    def gather_wait(b):
        pltpu.make_async_copy(table_hbm.at[idxt_v.at[0, 0]], rows_v.at[b],
                              gsem.at[b]).wait()

    def store_start(tt, j, b):
        for a in range(4):
            pltpu.async_copy(tbuf_v.at[b, pl.ds(8 * a, 8)],
                             out_hbm.at[j, a, TBLK * wid + tt], ssem.at[b])

    def store_wait(b):
        for a in range(4):
            pltpu.make_async_copy(tbuf_v.at[b, pl.ds(8 * a, 8)],
                                  out_hbm.at[0, a, 0], ssem.at[b]).wait()

    def unit(u):
        tt = u // SLEN
        j = u - tt * SLEN
        return tt, j

    for b in range(NBUF):
        tt, j = unit(b)
        gather_start(tt, j, b)

    @pl.loop(0, NROUNDS)
    def _round(r):
        for b in range(NBUF):
            u = r * NBUF + b
            tt, j = unit(u)
            gather_wait(b)

            @pl.when(u >= NBUF)
            def _():
                store_wait(b)

            @pl.loop(0, DIM, unroll=8)
            def _tr(k):
                kc = (k + lane) & (DIM - 1)
                for m in range(8):
                    rows = lane + 16 * m
                    vals = plsc.load_gather(rows_v.at[b], [rows, kc])
                    plsc.store_scatter(tbuf_v.at[b], [kc, rows], vals)

            store_start(tt, j, b)
            nxt = u + NBUF

            @pl.when(nxt < UNITS)
            def _():
                tt2 = nxt // SLEN
                j2 = nxt - tt2 * SLEN
                gather_start(tt2, j2, b)

    for b in range(NBUF):
        store_wait(b)


def _wt_body(x_ref, o_ref):
    xT = x_ref[...].T
    q = lax.broadcasted_iota(jnp.int32, (32, 128), 0)
    c = lax.broadcasted_iota(jnp.int32, (32, 128), 1)
    for j in range(4):
        pjt = (c == 4 * q + j).astype(jnp.float32)
        for u in range(8):
            xu = xT[128 * u:128 * (u + 1)]
            sel = jnp.dot(pjt, xu, preferred_element_type=jnp.float32)
            o_ref[32 * u:32 * (u + 1), 32 * j:32 * (j + 1)] = sel


def _to_row_major(weight):
    wt = weight.T
    w128 = pl.pallas_call(
        _wt_body,
        grid=(977,),
        in_specs=[pl.BlockSpec((32, 1024), lambda g: (0, g))],
        out_specs=pl.BlockSpec((256, 128), lambda g: (g, 0)),
        out_shape=jax.ShapeDtypeStruct((250000, 128), jnp.float32),
    )(wt)
    return w128.reshape(NUM_ROWS, DIM)


@jax.jit
def _embed(idx, weight):
    mesh = plsc.VectorSubcoreMesh(core_axis_name="c", subcore_axis_name="s")
    run = pl.kernel(
        _embed_body,
        out_type=jax.ShapeDtypeStruct((SLEN, 4, SEQ // 128, 8, 128),
                                      jnp.float32),
        mesh=mesh,
        compiler_params=pltpu.CompilerParams(
            use_tc_tiling_on_sc=False, needs_layout_passes=False
        ),
        scratch_types=[
            pltpu.VMEM((SENT_PER_W, SLEN), jnp.int32),
            pltpu.VMEM((TBLK, SLEN, 128), jnp.int32),
            pltpu.VMEM((NBUF, 128, DIM), jnp.float32),
            pltpu.VMEM((NBUF, DIM, 128), jnp.float32),
            pltpu.SemaphoreType.DMA((NBUF,)),
            pltpu.SemaphoreType.DMA((NBUF,)),
        ],
    )
    ot = run(idx, _to_row_major(weight))
    return ot.transpose(2, 4, 0, 1, 3).reshape(SEQ, SLEN, DIM)


def kernel(input, weight):
    return _embed(input.astype(jnp.int32), weight)


# speedup vs baseline: 5.5963x; 1.2949x over previous
import jax
import jax.numpy as jnp
from jax import lax
from jax.experimental import pallas as pl
from jax.experimental.pallas import tpu as pltpu
from jax.experimental.pallas import tpu_sc as plsc

NUM_ROWS = 1000000
DIM = 32
SEQ = 16384
SLEN = 50
NC, NS = 2, 16
NW = NC * NS
SENT_PER_W = SEQ // NW
TBLK = 4
NBUF = 8
UNITS = TBLK * SLEN
NROUNDS = UNITS // NBUF


def _embed_body(idx_hbm, table_hbm, out_hbm, idx_v, idxt_v, rows_v, tbuf_v,
                gsem, ssem):
    wid = lax.axis_index("s") * NC + lax.axis_index("c")
    s0 = wid * SENT_PER_W

    pltpu.sync_copy(idx_hbm.at[pl.ds(s0, SENT_PER_W)], idx_v)

    lane = lax.iota(jnp.int32, 16)

    @pl.loop(0, SLEN)
    def _build(j):
        col = jnp.full((16,), 0, jnp.int32) + j
        for tt in range(TBLK):
            for m in range(8):
                rows = lane + (128 * tt + 16 * m)
                vals = plsc.load_gather(idx_v, [rows, col])
                idxt_v[tt, j, pl.ds(16 * m, 16)] = vals

    def gather_start(tt, j, b):
        pltpu.async_copy(table_hbm.at[idxt_v.at[tt, j]], rows_v.at[b],
                         gsem.at[b])

    def gather_wait(b):
        pltpu.make_async_copy(table_hbm.at[idxt_v.at[0, 0]], rows_v.at[b],
                              gsem.at[b]).wait()

    def store_start(tt, j, b):
        for a in range(4):
            pltpu.async_copy(tbuf_v.at[b, pl.ds(8 * a, 8)],
                             out_hbm.at[j, a, TBLK * wid + tt], ssem.at[b])

    def store_wait(b):
        for a in range(4):
            pltpu.make_async_copy(tbuf_v.at[b, pl.ds(8 * a, 8)],
                                  out_hbm.at[0, a, 0], ssem.at[b]).wait()

    def unit(u):
        tt = u // SLEN
        j = u - tt * SLEN
        return tt, j

    for b in range(NBUF):
        tt, j = unit(b)
        gather_start(tt, j, b)

    @pl.loop(0, NROUNDS)
    def _round(r):
        for b in range(NBUF):
            u = r * NBUF + b
            tt, j = unit(u)
            gather_wait(b)

            @pl.when(u >= NBUF)
            def _():
                store_wait(b)

            @pl.loop(0, DIM, unroll=8)
            def _tr(k):
                kc = (k + lane) & (DIM - 1)
                for m in range(8):
                    rows = lane + 16 * m
                    vals = plsc.load_gather(rows_v.at[b], [rows, kc])
                    plsc.store_scatter(tbuf_v.at[b], [kc, rows], vals)

            store_start(tt, j, b)
            nxt = u + NBUF

            @pl.when(nxt < UNITS)
            def _():
                tt2 = nxt // SLEN
                j2 = nxt - tt2 * SLEN
                gather_start(tt2, j2, b)

    for b in range(NBUF):
        store_wait(b)


@jax.jit
def _embed(idx, weight):
    mesh = plsc.VectorSubcoreMesh(core_axis_name="c", subcore_axis_name="s")
    run = pl.kernel(
        _embed_body,
        out_type=jax.ShapeDtypeStruct((SLEN, 4, SEQ // 128, 8, 128),
                                      jnp.float32),
        mesh=mesh,
        compiler_params=pltpu.CompilerParams(
            use_tc_tiling_on_sc=False, needs_layout_passes=False
        ),
        scratch_types=[
            pltpu.VMEM((SENT_PER_W, SLEN), jnp.int32),
            pltpu.VMEM((TBLK, SLEN, 128), jnp.int32),
            pltpu.VMEM((NBUF, 128, DIM), jnp.float32),
            pltpu.VMEM((NBUF, DIM, 128), jnp.float32),
            pltpu.SemaphoreType.DMA((NBUF,)),
            pltpu.SemaphoreType.DMA((NBUF,)),
        ],
    )
    ot = run(idx, weight)
    return ot.transpose(2, 4, 0, 1, 3).reshape(SEQ, SLEN, DIM)


def kernel(input, weight):
    return _embed(input.astype(jnp.int32), weight)


# speedup vs baseline: 7.8294x; 1.3990x over previous
import jax
import jax.numpy as jnp
from jax import lax
from jax.experimental import pallas as pl
from jax.experimental.pallas import tpu as pltpu
from jax.experimental.pallas import tpu_sc as plsc

NUM_ROWS = 1000000
DIM = 32
SEQ = 16384
SLEN = 50
NC, NS = 2, 16
NW = NC * NS
SENT_PER_W = SEQ // NW
TBLK = 4
NBUF = 4
UNITS = TBLK * SLEN
NROUNDS = UNITS // NBUF


def _embed_body(idx_hbm, table_hbm, out_hbm, idx_v, idxt_v, rows_v, tbuf_v,
                gsem, ssem):
    wid = lax.axis_index("s") * NC + lax.axis_index("c")
    s0 = wid * SENT_PER_W

    pltpu.sync_copy(idx_hbm.at[pl.ds(s0, SENT_PER_W)], idx_v)

    lane = lax.iota(jnp.int32, 16)

    @pl.loop(0, SLEN)
    def _build(j):
        col = jnp.full((16,), 0, jnp.int32) + j
        for tt in range(TBLK):
            for m in range(8):
                rows = lane + (128 * tt + 16 * m)
                vals = plsc.load_gather(idx_v, [rows, col])
                idxt_v[tt, j, pl.ds(16 * m, 16)] = vals

    def gather_start(tt, j, b):
        pltpu.async_copy(table_hbm.at[idxt_v.at[tt, j]], rows_v.at[b],
                         gsem.at[b])

    def gather_wait(b):
        pltpu.make_async_copy(table_hbm.at[idxt_v.at[0, 0]], rows_v.at[b],
                              gsem.at[b]).wait()

    def store_start(tt, j, b):
        for a in range(4):
            pltpu.async_copy(tbuf_v.at[b, pl.ds(8 * a, 8)],
                             out_hbm.at[j, a, TBLK * wid + tt], ssem.at[b])

    def store_wait(b):
        for a in range(4):
            pltpu.make_async_copy(tbuf_v.at[b, pl.ds(8 * a, 8)],
                                  out_hbm.at[0, a, 0], ssem.at[b]).wait()

    def unit(u):
        tt = u // SLEN
        j = u - tt * SLEN
        return tt, j

    for b in range(NBUF):
        tt, j = unit(b)
        gather_start(tt, j, b)

    @pl.loop(0, NROUNDS)
    def _round(r):
        for b in range(NBUF):
            u = r * NBUF + b
            tt, j = unit(u)
            gather_wait(b)

            @pl.when(u >= NBUF)
            def _():
                store_wait(b)

            @plsc.parallel_loop(0, DIM, step=1, unroll=8)
            def _tr(k):
                kc = (k + lane) & (DIM - 1)
                for m in range(8):
                    rows = lane + 16 * m
                    vals = plsc.load_gather(rows_v.at[b], [rows, kc])
                    plsc.store_scatter(tbuf_v.at[b], [kc, rows], vals)

            store_start(tt, j, b)
            nxt = u + NBUF

            @pl.when(nxt < UNITS)
            def _():
                tt2 = nxt // SLEN
                j2 = nxt - tt2 * SLEN
                gather_start(tt2, j2, b)

    for b in range(NBUF):
        store_wait(b)


@jax.jit
def _embed(idx, weight):
    mesh = plsc.VectorSubcoreMesh(core_axis_name="c", subcore_axis_name="s")
    run = pl.kernel(
        _embed_body,
        out_type=jax.ShapeDtypeStruct((SLEN, 4, SEQ // 128, 8, 128),
                                      jnp.float32),
        mesh=mesh,
        compiler_params=pltpu.CompilerParams(
            use_tc_tiling_on_sc=False, needs_layout_passes=False
        ),
        scratch_types=[
            pltpu.VMEM((SENT_PER_W, SLEN), jnp.int32),
            pltpu.VMEM((TBLK, SLEN, 128), jnp.int32),
            pltpu.VMEM((NBUF, 128, DIM), jnp.float32),
            pltpu.VMEM((NBUF, DIM, 128), jnp.float32),
            pltpu.SemaphoreType.DMA((NBUF,)),
            pltpu.SemaphoreType.DMA((NBUF,)),
        ],
    )
    ot = run(idx, weight)
    return ot.transpose(2, 4, 0, 1, 3).reshape(SEQ, SLEN, DIM)


def kernel(input, weight):
    return _embed(input.astype(jnp.int32), weight)
